# baseline (speedup 1.0000x reference)
import functools

import jax
import jax.numpy as jnp
from jax import lax
from jax.experimental import pallas as pl
from jax.experimental.pallas import tpu as pltpu
from jax.experimental.pallas import tpu_sc as plsc

N = 10000
IN_DIM = 128
EMB = 64
HID = 64
E_IN = 160000
E_TOT = E_IN + N

NC = 2
NS = 16
NW = NC * NS
ROW = 80
BLK = 128
SUB = 2
EPB = SUB * BLK
BPW = 2 * -(-E_TOT // (NW * EPB * 2))
EPW = BPW * EPB
E_PAD = EPW * NW
ZR = 632
N_PAD = NS * ZR

F32 = jnp.float32


def _enc_body(x_ref, ws_ref, bs_ref, wg_ref, asrc_ref, adst_ref,
              wa1_ref, ba1_ref, wa2_ref, ba2_ref,
              hpext_ref, as_ref, ad_ref, m_ref, xa_ref, ms_ref, md_ref,
              acc_ref):
    i = pl.program_id(0)
    h = jnp.maximum(jnp.dot(x_ref[...], ws_ref[...],
                            preferred_element_type=F32) + bs_ref[...], 0.0)
    hp = jnp.dot(h, wg_ref[...], preferred_element_type=F32)
    a_s = jnp.dot(hp, asrc_ref[...], preferred_element_type=F32)
    a_d = jnp.dot(hp, adst_ref[...], preferred_element_type=F32)
    blk = hp.shape[0]
    ones = jnp.ones((blk, 1), F32)
    zeros = jnp.zeros((blk, ROW - EMB - 1), F32)
    hpext_ref[...] = jnp.concatenate([hp, ones, zeros], axis=1)
    as_ref[...] = a_s
    ad_ref[...] = a_d

    @pl.when(i == 0)
    def _():
        ms_ref[0] = -jnp.inf
        md_ref[0] = -jnp.inf

    ms_ref[0] = jnp.maximum(ms_ref[0], jnp.max(a_s))
    md_ref[0] = jnp.maximum(md_ref[0], jnp.max(a_d))
    m_ref[...] = jnp.full((1, 128), jnp.maximum(ms_ref[0] + md_ref[0], 0.0), F32)

    @pl.when(i == 0)
    def _():
        acc_ref[...] = jnp.zeros_like(acc_ref)

    acc_ref[...] += lax.dot_general(x_ref[...], wa1_ref[...],
                                    (((0,), (0,)), ((), ())),
                                    preferred_element_type=F32)

    @pl.when(i == pl.num_programs(0) - 1)
    def _():
        xa1 = jnp.maximum(acc_ref[...] + ba1_ref[...], 0.0)
        xa_ref[...] = jnp.dot(xa1, wa2_ref[...],
                              preferred_element_type=F32) + ba2_ref[...]


def _tc_encode(x, W_stru, b_stru, W_gat, att_src, att_dst,
               W_a1, b_a1, W_a2, b_a2):
    blk = 2000
    grid = N // blk
    return pl.pallas_call(
        _enc_body,
        grid=(grid,),
        in_specs=[
            pl.BlockSpec((blk, IN_DIM), lambda i: (i, 0)),
            pl.BlockSpec((IN_DIM, EMB), lambda i: (0, 0)),
            pl.BlockSpec((1, EMB), lambda i: (0, 0)),
            pl.BlockSpec((EMB, HID), lambda i: (0, 0)),
            pl.BlockSpec((HID, 1), lambda i: (0, 0)),
            pl.BlockSpec((HID, 1), lambda i: (0, 0)),
            pl.BlockSpec((blk, EMB), lambda i: (i, 0)),
            pl.BlockSpec((1, EMB), lambda i: (0, 0)),
            pl.BlockSpec((EMB, HID), lambda i: (0, 0)),
            pl.BlockSpec((1, HID), lambda i: (0, 0)),
        ],
        out_specs=[
            pl.BlockSpec((blk, ROW), lambda i: (i, 0)),
            pl.BlockSpec((blk, 1), lambda i: (i, 0)),
            pl.BlockSpec((blk, 1), lambda i: (i, 0)),
            pl.BlockSpec((1, 128), lambda i: (0, 0)),
            pl.BlockSpec((IN_DIM, HID), lambda i: (0, 0)),
        ],
        out_shape=[
            jax.ShapeDtypeStruct((N, ROW), F32),
            jax.ShapeDtypeStruct((N, 1), F32),
            jax.ShapeDtypeStruct((N, 1), F32),
            jax.ShapeDtypeStruct((1, 128), F32),
            jax.ShapeDtypeStruct((IN_DIM, HID), F32),
        ],
        scratch_shapes=[pltpu.SMEM((1,), F32), pltpu.SMEM((1,), F32),
                        pltpu.VMEM((IN_DIM, EMB), F32)],
    )(x, W_stru, b_stru, W_gat, att_src, att_dst, W_a1, b_a1, W_a2, b_a2)


NBUF = 2


def _sc_edge_body(hp_hbm, asrc_hbm, adst_hbm, src_hbm, dst_hbm, m_hbm, z_hbm,
                  out_hbm,
                  a_s_v, a_d_v, m_v, idx_s_all, idx_d_all,
                  r0, r1, g0, g1, s0, s1, u_sh):
    cid = lax.axis_index("c")
    sid = lax.axis_index("s")
    wid = sid * NC + cid

    rows = (r0, r1)
    gsem = (g0, g1)
    ssem = (s0, s1)

    pltpu.sync_copy(asrc_hbm, a_s_v)
    pltpu.sync_copy(adst_hbm, a_d_v)
    pltpu.sync_copy(m_hbm, m_v)
    pltpu.sync_copy(src_hbm.at[wid], idx_s_all)
    pltpu.sync_copy(dst_hbm.at[wid], idx_d_all)
    pltpu.sync_copy(z_hbm, u_sh.at[pl.ds(sid * ZR, ZR)])
    plsc.subcore_barrier()

    def start_gather(q, b):
        for j in range(SUB):
            pltpu.async_copy(hp_hbm.at[idx_s_all.at[b, j]],
                             rows[q].at[pl.ds(j * BLK, BLK)], gsem[q])

    def drain_scatter(q):
        for j in range(SUB):
            pltpu.make_async_copy(rows[q].at[pl.ds(j * BLK, BLK)],
                                  u_sh.at[idx_d_all.at[0, 0]], ssem[q]).wait()

    def process(q, b):
        for j in range(SUB):
            pltpu.make_async_copy(hp_hbm.at[idx_s_all.at[b, j]],
                                  rows[q].at[pl.ds(j * BLK, BLK)],
                                  gsem[q]).wait()
        base = wid * EPW + b * EPB
        m_vec = m_v[...]
        rq = rows[q]
        for g in range(EPB // 16):
            s_i = idx_s_all[b, g // 8, pl.ds((g % 8) * 16, 16)]
            d_i = idx_d_all[b, g // 8, pl.ds((g % 8) * 16, 16)]
            av = plsc.load_gather(a_s_v, [s_i])
            bv = plsc.load_gather(a_d_v, [d_i])
            e = av + bv
            e = jnp.where(e >= 0.0, e, e * jnp.float32(0.2))
            w = jnp.exp(e - m_vec)
            gids = base + g * 16 + lax.iota(jnp.int32, 16)
            w = jnp.where(gids < E_TOT, w, 0.0)
            for lane in range(16):
                i = g * 16 + lane
                ws = w[lane]
                for ch in range(ROW // 16):
                    rq[i, pl.ds(ch * 16, 16)] = rq[i, pl.ds(ch * 16, 16)] * ws
        for j in range(SUB):
            pltpu.async_copy(rows[q].at[pl.ds(j * BLK, BLK)],
                             u_sh.at[idx_d_all.at[b, j]], ssem[q], add=True)

    start_gather(0, 0)

    def outer_body(i, carry):
        for q in range(NBUF):
            b = i * NBUF + q
            nq = 1 - q
            if q == 0:
                @pl.when(i > 0)
                def _():
                    drain_scatter(nq)

                start_gather(nq, b + 1)
            else:
                @pl.when(i < BPW // NBUF - 1)
                def _():
                    drain_scatter(nq)
                    start_gather(nq, b + 1)
            process(q, b)
        return carry

    lax.fori_loop(0, BPW // NBUF, outer_body, 0)
    for q in range(NBUF):
        drain_scatter(q)
    plsc.subcore_barrier()
    pltpu.sync_copy(u_sh.at[pl.ds(sid * ZR, ZR)],
                    out_hbm.at[cid, pl.ds(sid * ZR, ZR)])


def _sc_edge(hp_ext, a_src, a_dst, src, dst, m8, zeros_tile):
    mesh = plsc.VectorSubcoreMesh(core_axis_name="c", subcore_axis_name="s",
                                  num_cores=NC, num_subcores=NS)
    return pl.kernel(
        _sc_edge_body,
        out_type=jax.ShapeDtypeStruct((NC, N_PAD, ROW), F32),
        mesh=mesh,
        scratch_types=(
            [pltpu.VMEM((N,), F32),
             pltpu.VMEM((N,), F32),
             pltpu.VMEM((16,), F32)]
            + [pltpu.VMEM((BPW, SUB, BLK), jnp.int32)] * 2
            + [pltpu.VMEM((EPB, ROW), F32)] * NBUF
            + [pltpu.SemaphoreType.DMA] * (2 * NBUF)
            + [pltpu.VMEM_SHARED((N_PAD, ROW), F32)]
        ),
        compiler_params=pltpu.CompilerParams(needs_layout_passes=False, use_tc_tiling_on_sc=False),
    )(hp_ext, a_src, a_dst, src, dst, m8, zeros_tile)


def _mid_body(u_ref, bg_ref, emb_ref):
    u = u_ref[0] + u_ref[1]
    emb_ref[...] = u[:, :EMB] / (u[:, EMB:EMB + 1] + 1e-16) + bg_ref[...]


def _tc_mid(uext, b_gat):
    blk = 2000
    grid = N // blk
    return pl.pallas_call(
        _mid_body,
        grid=(grid,),
        in_specs=[
            pl.BlockSpec((NC, blk, ROW), lambda i: (0, i, 0)),
            pl.BlockSpec((1, EMB), lambda i: (0, 0)),
        ],
        out_specs=pl.BlockSpec((blk, EMB), lambda i: (i, 0)),
        out_shape=jax.ShapeDtypeStruct((N, EMB), F32),
    )(uext, b_gat)


def _big_body(embi_ref, embj_ref, xa_ref, x_ref, s_ref):
    j = pl.program_id(1)
    s_ref[...] = jax.nn.sigmoid(
        lax.dot_general(embi_ref[...], embj_ref[...],
                        (((1,), (1,)), ((), ())), preferred_element_type=F32))

    @pl.when(j == 0)
    def _():
        x_ref[...] = lax.dot_general(embi_ref[...], xa_ref[...],
                                     (((1,), (1,)), ((), ())),
                                     preferred_element_type=F32)


def _tc_big(emb, xa):
    bi = 2000
    bj = 1024
    gi = N // bi
    gj = pl.cdiv(N, bj)
    return pl.pallas_call(
        _big_body,
        grid=(gi, gj),
        in_specs=[
            pl.BlockSpec((bi, EMB), lambda i, j: (i, 0)),
            pl.BlockSpec((bj, EMB), lambda i, j: (j, 0)),
            pl.BlockSpec((IN_DIM, HID), lambda i, j: (0, 0)),
        ],
        out_specs=[
            pl.BlockSpec((bi, IN_DIM), lambda i, j: (i, 0)),
            pl.BlockSpec((bi, bj), lambda i, j: (i, j)),
        ],
        out_shape=[
            jax.ShapeDtypeStruct((N, IN_DIM), F32),
            jax.ShapeDtypeStruct((N, N), F32),
        ],
    )(emb, emb, xa)


def kernel(x, edge_index, batch_size, W_stru, b_stru, W_gat, att_src,
           att_dst, b_gat, W_a1, b_a1, W_a2, b_a2):
    x = x.astype(F32)
    hp_ext, a_s2, a_d2, m_out, xa = _tc_encode(
        x, W_stru, b_stru.reshape(1, EMB), W_gat,
        att_src.reshape(HID, 1), att_dst.reshape(HID, 1),
        W_a1, b_a1.reshape(1, EMB), W_a2, b_a2.reshape(1, HID))

    loops = jnp.arange(N, dtype=jnp.int32)
    pad = jnp.zeros((E_PAD - E_TOT,), jnp.int32)
    src = jnp.concatenate(
        [edge_index[0].astype(jnp.int32), loops, pad]
    ).reshape(NW, BPW, SUB, BLK)
    dst = jnp.concatenate(
        [edge_index[1].astype(jnp.int32), loops, pad]
    ).reshape(NW, BPW, SUB, BLK)

    uext = _sc_edge(hp_ext, a_s2.reshape(N), a_d2.reshape(N), src, dst,
                    m_out[0, :16], jnp.zeros((ZR, ROW), F32))

    uext = uext[:, :N]
    emb = _tc_mid(uext, b_gat.reshape(1, HID))
    x_, s_ = _tc_big(emb, xa)
    return (x_, s_)


# speedup vs baseline: 1.4384x; 1.4384x over previous
import functools

import jax
import jax.numpy as jnp
from jax import lax
from jax.experimental import pallas as pl
from jax.experimental.pallas import tpu as pltpu
from jax.experimental.pallas import tpu_sc as plsc

N = 10000
IN_DIM = 128
EMB = 64
HID = 64
E_IN = 160000
E_TOT = E_IN + N

NC = 2
NS = 16
NW = NC * NS
ROW = 80
BLK = 128
BPW = -(-E_TOT // (NW * BLK))
EPW = BPW * BLK
E_PAD = EPW * NW
ZR = 632
N_PAD = NS * ZR

F32 = jnp.float32


def _enc_body(x_ref, ws_ref, bs_ref, wg_ref, asrc_ref, adst_ref,
              wa1_ref, ba1_ref, wa2_ref, ba2_ref,
              hpext_ref, as_ref, ad_ref, m_ref, xa_ref, ms_ref, md_ref,
              acc_ref):
    i = pl.program_id(0)
    h = jnp.maximum(jnp.dot(x_ref[...], ws_ref[...],
                            preferred_element_type=F32) + bs_ref[...], 0.0)
    hp = jnp.dot(h, wg_ref[...], preferred_element_type=F32)
    a_s = jnp.dot(hp, asrc_ref[...], preferred_element_type=F32)
    a_d = jnp.dot(hp, adst_ref[...], preferred_element_type=F32)
    blk = hp.shape[0]
    ones = jnp.ones((blk, 1), F32)
    zeros = jnp.zeros((blk, ROW - EMB - 1), F32)
    hpext_ref[...] = jnp.concatenate([hp, ones, zeros], axis=1)
    as_ref[...] = a_s
    ad_ref[...] = a_d

    @pl.when(i == 0)
    def _():
        ms_ref[0] = -jnp.inf
        md_ref[0] = -jnp.inf

    ms_ref[0] = jnp.maximum(ms_ref[0], jnp.max(a_s))
    md_ref[0] = jnp.maximum(md_ref[0], jnp.max(a_d))
    m_ref[...] = jnp.full((1, 128), jnp.maximum(ms_ref[0] + md_ref[0], 0.0), F32)

    @pl.when(i == 0)
    def _():
        acc_ref[...] = jnp.zeros_like(acc_ref)

    acc_ref[...] += lax.dot_general(x_ref[...], wa1_ref[...],
                                    (((0,), (0,)), ((), ())),
                                    preferred_element_type=F32)

    @pl.when(i == pl.num_programs(0) - 1)
    def _():
        xa1 = jnp.maximum(acc_ref[...] + ba1_ref[...], 0.0)
        xa_ref[...] = jnp.dot(xa1, wa2_ref[...],
                              preferred_element_type=F32) + ba2_ref[...]


def _tc_encode(x, W_stru, b_stru, W_gat, att_src, att_dst,
               W_a1, b_a1, W_a2, b_a2):
    blk = 2000
    grid = N // blk
    return pl.pallas_call(
        _enc_body,
        grid=(grid,),
        in_specs=[
            pl.BlockSpec((blk, IN_DIM), lambda i: (i, 0)),
            pl.BlockSpec((IN_DIM, EMB), lambda i: (0, 0)),
            pl.BlockSpec((1, EMB), lambda i: (0, 0)),
            pl.BlockSpec((EMB, HID), lambda i: (0, 0)),
            pl.BlockSpec((HID, 1), lambda i: (0, 0)),
            pl.BlockSpec((HID, 1), lambda i: (0, 0)),
            pl.BlockSpec((blk, EMB), lambda i: (i, 0)),
            pl.BlockSpec((1, EMB), lambda i: (0, 0)),
            pl.BlockSpec((EMB, HID), lambda i: (0, 0)),
            pl.BlockSpec((1, HID), lambda i: (0, 0)),
        ],
        out_specs=[
            pl.BlockSpec((blk, ROW), lambda i: (i, 0)),
            pl.BlockSpec((blk, 1), lambda i: (i, 0)),
            pl.BlockSpec((blk, 1), lambda i: (i, 0)),
            pl.BlockSpec((1, 128), lambda i: (0, 0)),
            pl.BlockSpec((IN_DIM, HID), lambda i: (0, 0)),
        ],
        out_shape=[
            jax.ShapeDtypeStruct((N, ROW), F32),
            jax.ShapeDtypeStruct((N, 1), F32),
            jax.ShapeDtypeStruct((N, 1), F32),
            jax.ShapeDtypeStruct((1, 128), F32),
            jax.ShapeDtypeStruct((IN_DIM, HID), F32),
        ],
        scratch_shapes=[pltpu.SMEM((1,), F32), pltpu.SMEM((1,), F32),
                        pltpu.VMEM((IN_DIM, EMB), F32)],
    )(x, W_stru, b_stru, W_gat, att_src, att_dst, W_a1, b_a1, W_a2, b_a2)


NBUF = 3


def _sc_edge_body(hp_hbm, asrc_hbm, adst_hbm, src_hbm, dst_hbm, m_hbm, z_hbm,
                  out_hbm,
                  a_s_v, a_d_v, m_v, idx_s_all, idx_d_all,
                  r0, r1, r2, g0, g1, g2, s0, s1, s2, u_sh):
    cid = lax.axis_index("c")
    sid = lax.axis_index("s")
    wid = sid * NC + cid

    rows = (r0, r1, r2)
    gsem = (g0, g1, g2)
    ssem = (s0, s1, s2)

    pltpu.sync_copy(asrc_hbm, a_s_v)
    pltpu.sync_copy(adst_hbm, a_d_v)
    pltpu.sync_copy(m_hbm, m_v)
    pltpu.sync_copy(src_hbm.at[wid], idx_s_all)
    pltpu.sync_copy(dst_hbm.at[wid], idx_d_all)
    pltpu.sync_copy(z_hbm, u_sh.at[pl.ds(sid * ZR, ZR)])
    plsc.subcore_barrier()

    def start_gather(q, b):
        pltpu.async_copy(hp_hbm.at[idx_s_all.at[b]], rows[q], gsem[q])

    def drain_scatter(q):
        pltpu.make_async_copy(rows[q], u_sh.at[idx_d_all.at[0]], ssem[q]).wait()

    def process(q, b):
        pltpu.make_async_copy(hp_hbm.at[idx_s_all.at[b]], rows[q], gsem[q]).wait()
        m_vec = m_v[...]
        rq = rows[q]
        for g in range(BLK // 16):
            s_i = idx_s_all[b, pl.ds(g * 16, 16)]
            d_i = idx_d_all[b, pl.ds(g * 16, 16)]
            av = plsc.load_gather(a_s_v, [s_i])
            bv = plsc.load_gather(a_d_v, [d_i])
            e = av + bv
            e = jnp.maximum(e, e * jnp.float32(0.2))
            w = jnp.exp(e - m_vec)
            for lane in range(16):
                i = g * 16 + lane
                ws = w[lane]
                for ch in range(ROW // 16):
                    rq[i, pl.ds(ch * 16, 16)] = rq[i, pl.ds(ch * 16, 16)] * ws
        pltpu.async_copy(rq, u_sh.at[idx_d_all.at[b]], ssem[q], add=True)

    start_gather(0, 0)
    start_gather(1, 1)

    def outer_body(i, carry):
        for q in range(NBUF):
            b = i * NBUF + q
            process(q, b)
            nq = (q + 2) % NBUF
            if q == 0:
                @pl.when(i > 0)
                def _():
                    drain_scatter(nq)

                start_gather(nq, b + 2)
            else:
                @pl.when(i < BPW // NBUF - 1)
                def _():
                    drain_scatter(nq)
                    start_gather(nq, b + 2)
        return carry

    lax.fori_loop(0, BPW // NBUF, outer_body, 0)
    for q in range(NBUF):
        drain_scatter(q)
    plsc.subcore_barrier()
    pltpu.sync_copy(u_sh.at[pl.ds(sid * ZR, ZR)],
                    out_hbm.at[cid, pl.ds(sid * ZR, ZR)])


def _sc_edge(hp_ext, a_src, a_dst, src, dst, m8, zeros_tile):
    mesh = plsc.VectorSubcoreMesh(core_axis_name="c", subcore_axis_name="s",
                                  num_cores=NC, num_subcores=NS)
    return pl.kernel(
        _sc_edge_body,
        out_type=jax.ShapeDtypeStruct((NC, N_PAD, ROW), F32),
        mesh=mesh,
        scratch_types=(
            [pltpu.VMEM((N + 16,), F32),
             pltpu.VMEM((N + 16,), F32),
             pltpu.VMEM((16,), F32)]
            + [pltpu.VMEM((BPW, BLK), jnp.int32)] * 2
            + [pltpu.VMEM((BLK, ROW), F32)] * NBUF
            + [pltpu.SemaphoreType.DMA] * (2 * NBUF)
            + [pltpu.VMEM_SHARED((N_PAD, ROW), F32)]
        ),
        compiler_params=pltpu.CompilerParams(needs_layout_passes=False, use_tc_tiling_on_sc=False),
    )(hp_ext, a_src, a_dst, src, dst, m8, zeros_tile)


def _mid_body(u_ref, bg_ref, emb_ref):
    u = u_ref[0] + u_ref[1]
    emb_ref[...] = u[:, :EMB] / (u[:, EMB:EMB + 1] + 1e-16) + bg_ref[...]


def _tc_mid(uext, b_gat):
    blk = 2000
    grid = N // blk
    return pl.pallas_call(
        _mid_body,
        grid=(grid,),
        in_specs=[
            pl.BlockSpec((NC, blk, ROW), lambda i: (0, i, 0)),
            pl.BlockSpec((1, EMB), lambda i: (0, 0)),
        ],
        out_specs=pl.BlockSpec((blk, EMB), lambda i: (i, 0)),
        out_shape=jax.ShapeDtypeStruct((N, EMB), F32),
    )(uext, b_gat)


def _big_body(embi_ref, embj_ref, xa_ref, x_ref, s_ref):
    j = pl.program_id(1)
    s_ref[...] = jax.nn.sigmoid(
        lax.dot_general(embi_ref[...], embj_ref[...],
                        (((1,), (1,)), ((), ())), preferred_element_type=F32))

    @pl.when(j == 0)
    def _():
        x_ref[...] = lax.dot_general(embi_ref[...], xa_ref[...],
                                     (((1,), (1,)), ((), ())),
                                     preferred_element_type=F32)


def _tc_big(emb, xa):
    bi = 2000
    bj = 1024
    gi = N // bi
    gj = pl.cdiv(N, bj)
    return pl.pallas_call(
        _big_body,
        grid=(gi, gj),
        in_specs=[
            pl.BlockSpec((bi, EMB), lambda i, j: (i, 0)),
            pl.BlockSpec((bj, EMB), lambda i, j: (j, 0)),
            pl.BlockSpec((IN_DIM, HID), lambda i, j: (0, 0)),
        ],
        out_specs=[
            pl.BlockSpec((bi, IN_DIM), lambda i, j: (i, 0)),
            pl.BlockSpec((bi, bj), lambda i, j: (i, j)),
        ],
        out_shape=[
            jax.ShapeDtypeStruct((N, IN_DIM), F32),
            jax.ShapeDtypeStruct((N, N), F32),
        ],
    )(emb, emb, xa)


def kernel(x, edge_index, batch_size, W_stru, b_stru, W_gat, att_src,
           att_dst, b_gat, W_a1, b_a1, W_a2, b_a2):
    x = x.astype(F32)
    hp_ext, a_s2, a_d2, m_out, xa = _tc_encode(
        x, W_stru, b_stru.reshape(1, EMB), W_gat,
        att_src.reshape(HID, 1), att_dst.reshape(HID, 1),
        W_a1, b_a1.reshape(1, EMB), W_a2, b_a2.reshape(1, HID))

    loops = jnp.arange(N, dtype=jnp.int32)
    pad = jnp.full((E_PAD - E_TOT,), N, jnp.int32)
    src = jnp.concatenate(
        [edge_index[0].astype(jnp.int32), loops, pad]).reshape(NW, BPW, BLK)
    dst = jnp.concatenate(
        [edge_index[1].astype(jnp.int32), loops, pad]).reshape(NW, BPW, BLK)
    hp_ext = jnp.concatenate([hp_ext, jnp.zeros((16, ROW), F32)])
    a_pad = jnp.zeros((16,), F32)

    uext = _sc_edge(hp_ext, jnp.concatenate([a_s2.reshape(N), a_pad]),
                    jnp.concatenate([a_d2.reshape(N), a_pad]), src, dst,
                    m_out[0, :16], jnp.zeros((ZR, ROW), F32))

    uext = uext[:, :N]
    emb = _tc_mid(uext, b_gat.reshape(1, HID))
    x_, s_ = _tc_big(emb, xa)
    return (x_, s_)


# speedup vs baseline: 1.5202x; 1.0569x over previous
import functools

import jax
import jax.numpy as jnp
from jax import lax
from jax.experimental import pallas as pl
from jax.experimental.pallas import tpu as pltpu
from jax.experimental.pallas import tpu_sc as plsc

N = 10000
IN_DIM = 128
EMB = 64
HID = 64
E_IN = 160000
E_TOT = E_IN + N

NC = 2
NS = 16
NW = NC * NS
ROW = 80
BLK = 128
BPW = -(-E_TOT // (NW * BLK))
EPW = BPW * BLK
E_PAD = EPW * NW
ZR = 632
N_PAD = NS * ZR

F32 = jnp.float32


def _enc_body(x_ref, ws_ref, bs_ref, wg_ref, asrc_ref, adst_ref,
              wa1_ref, ba1_ref, wa2_ref, ba2_ref,
              hpext_ref, as_ref, ad_ref, m_ref, xa_ref, ms_ref, md_ref,
              acc_ref):
    i = pl.program_id(0)
    h = jnp.maximum(jnp.dot(x_ref[...], ws_ref[...],
                            preferred_element_type=F32) + bs_ref[...], 0.0)
    hp = jnp.dot(h, wg_ref[...], preferred_element_type=F32)
    a_s = jnp.dot(hp, asrc_ref[...], preferred_element_type=F32)
    a_d = jnp.dot(hp, adst_ref[...], preferred_element_type=F32)
    blk = hp.shape[0]
    ones = jnp.ones((blk, 1), F32)
    zeros = jnp.zeros((blk, ROW - EMB - 1), F32)
    hpext_ref[...] = jnp.concatenate([hp, ones, zeros], axis=1)
    as_ref[...] = a_s
    ad_ref[...] = a_d

    @pl.when(i == 0)
    def _():
        ms_ref[0] = -jnp.inf
        md_ref[0] = -jnp.inf

    ms_ref[0] = jnp.maximum(ms_ref[0], jnp.max(a_s))
    md_ref[0] = jnp.maximum(md_ref[0], jnp.max(a_d))
    m_ref[...] = jnp.full((1, 128), jnp.maximum(ms_ref[0] + md_ref[0], 0.0), F32)

    @pl.when(i == 0)
    def _():
        acc_ref[...] = jnp.zeros_like(acc_ref)

    acc_ref[...] += lax.dot_general(x_ref[...], wa1_ref[...],
                                    (((0,), (0,)), ((), ())),
                                    preferred_element_type=F32)

    @pl.when(i == pl.num_programs(0) - 1)
    def _():
        xa1 = jnp.maximum(acc_ref[...] + ba1_ref[...], 0.0)
        xa_ref[...] = jnp.dot(xa1, wa2_ref[...],
                              preferred_element_type=F32) + ba2_ref[...]


def _tc_encode(x, W_stru, b_stru, W_gat, att_src, att_dst,
               W_a1, b_a1, W_a2, b_a2):
    blk = 2000
    grid = N // blk
    return pl.pallas_call(
        _enc_body,
        grid=(grid,),
        in_specs=[
            pl.BlockSpec((blk, IN_DIM), lambda i: (i, 0)),
            pl.BlockSpec((IN_DIM, EMB), lambda i: (0, 0)),
            pl.BlockSpec((1, EMB), lambda i: (0, 0)),
            pl.BlockSpec((EMB, HID), lambda i: (0, 0)),
            pl.BlockSpec((HID, 1), lambda i: (0, 0)),
            pl.BlockSpec((HID, 1), lambda i: (0, 0)),
            pl.BlockSpec((blk, EMB), lambda i: (i, 0)),
            pl.BlockSpec((1, EMB), lambda i: (0, 0)),
            pl.BlockSpec((EMB, HID), lambda i: (0, 0)),
            pl.BlockSpec((1, HID), lambda i: (0, 0)),
        ],
        out_specs=[
            pl.BlockSpec((blk, ROW), lambda i: (i, 0)),
            pl.BlockSpec((blk, 1), lambda i: (i, 0)),
            pl.BlockSpec((blk, 1), lambda i: (i, 0)),
            pl.BlockSpec((1, 128), lambda i: (0, 0)),
            pl.BlockSpec((IN_DIM, HID), lambda i: (0, 0)),
        ],
        out_shape=[
            jax.ShapeDtypeStruct((N, ROW), F32),
            jax.ShapeDtypeStruct((N, 1), F32),
            jax.ShapeDtypeStruct((N, 1), F32),
            jax.ShapeDtypeStruct((1, 128), F32),
            jax.ShapeDtypeStruct((IN_DIM, HID), F32),
        ],
        scratch_shapes=[pltpu.SMEM((1,), F32), pltpu.SMEM((1,), F32),
                        pltpu.VMEM((IN_DIM, EMB), F32)],
    )(x, W_stru, b_stru, W_gat, att_src, att_dst, W_a1, b_a1, W_a2, b_a2)


NBUF = 3


def _sc_edge_body(hp_hbm, asrc_hbm, adst_hbm, src_hbm, dst_hbm, m_hbm, z_hbm,
                  out_hbm,
                  a_s_v, a_d_v, m_v, idx_s_all, idx_d_all,
                  r0, r1, r2, g0, g1, g2, s0, s1, s2, u_sh):
    cid = lax.axis_index("c")
    sid = lax.axis_index("s")
    wid = sid * NC + cid

    rows = (r0, r1, r2)
    gsem = (g0, g1, g2)
    ssem = (s0, s1, s2)

    pltpu.sync_copy(asrc_hbm, a_s_v)
    pltpu.sync_copy(adst_hbm, a_d_v)
    pltpu.sync_copy(m_hbm, m_v)
    pltpu.sync_copy(src_hbm.at[wid], idx_s_all)
    pltpu.sync_copy(dst_hbm.at[wid], idx_d_all)
    pltpu.sync_copy(z_hbm, u_sh.at[pl.ds(sid * ZR, ZR)])
    plsc.subcore_barrier()

    def start_gather(q, b):
        pltpu.async_copy(hp_hbm.at[idx_s_all.at[b]], rows[q], gsem[q])

    def drain_scatter(q):
        pltpu.make_async_copy(rows[q], u_sh.at[idx_d_all.at[0]], ssem[q]).wait()

    def process(q, b):
        pltpu.make_async_copy(hp_hbm.at[idx_s_all.at[b]], rows[q], gsem[q]).wait()
        base = wid * EPW + b * BLK
        m_vec = m_v[...]
        rq = rows[q]
        for g in range(BLK // 16):
            s_i = idx_s_all[b, pl.ds(g * 16, 16)]
            d_i = idx_d_all[b, pl.ds(g * 16, 16)]
            av = plsc.load_gather(a_s_v, [s_i])
            bv = plsc.load_gather(a_d_v, [d_i])
            e = av + bv
            e = jnp.where(e >= 0.0, e, e * jnp.float32(0.2))
            w = jnp.exp(e - m_vec)
            gids = base + g * 16 + lax.iota(jnp.int32, 16)
            w = jnp.where(gids < E_TOT, w, 0.0)
            for lane in range(16):
                i = g * 16 + lane
                ws = w[lane]
                for ch in range(ROW // 16):
                    rq[i, pl.ds(ch * 16, 16)] = rq[i, pl.ds(ch * 16, 16)] * ws
        pltpu.async_copy(rq, u_sh.at[idx_d_all.at[b]], ssem[q], add=True)

    start_gather(0, 0)
    start_gather(1, 1)

    def outer_body(i, carry):
        for q in range(NBUF):
            b = i * NBUF + q
            process(q, b)
            nq = (q + 2) % NBUF
            if q == 0:
                @pl.when(i > 0)
                def _():
                    drain_scatter(nq)

                start_gather(nq, b + 2)
            else:
                @pl.when(i < BPW // NBUF - 1)
                def _():
                    drain_scatter(nq)
                    start_gather(nq, b + 2)
        return carry

    lax.fori_loop(0, BPW // NBUF, outer_body, 0)
    for q in range(NBUF):
        drain_scatter(q)
    plsc.subcore_barrier()
    pltpu.sync_copy(u_sh.at[pl.ds(sid * ZR, ZR)],
                    out_hbm.at[cid, pl.ds(sid * ZR, ZR)])


def _sc_edge(hp_ext, a_src, a_dst, src, dst, m8, zeros_tile):
    mesh = plsc.VectorSubcoreMesh(core_axis_name="c", subcore_axis_name="s",
                                  num_cores=NC, num_subcores=NS)
    return pl.kernel(
        _sc_edge_body,
        out_type=jax.ShapeDtypeStruct((NC, N_PAD, ROW), F32),
        mesh=mesh,
        scratch_types=(
            [pltpu.VMEM((N,), F32),
             pltpu.VMEM((N,), F32),
             pltpu.VMEM((16,), F32)]
            + [pltpu.VMEM((BPW, BLK), jnp.int32)] * 2
            + [pltpu.VMEM((BLK, ROW), F32)] * NBUF
            + [pltpu.SemaphoreType.DMA] * (2 * NBUF)
            + [pltpu.VMEM_SHARED((N_PAD, ROW), F32)]
        ),
        compiler_params=pltpu.CompilerParams(needs_layout_passes=False, use_tc_tiling_on_sc=False),
    )(hp_ext, a_src, a_dst, src, dst, m8, zeros_tile)


def _mid_body(u_ref, bg_ref, emb_ref):
    u = u_ref[0] + u_ref[1]
    emb_ref[...] = u[:, :EMB] / (u[:, EMB:EMB + 1] + 1e-16) + bg_ref[...]


def _tc_mid(uext, b_gat):
    blk = 2000
    grid = N // blk
    return pl.pallas_call(
        _mid_body,
        grid=(grid,),
        in_specs=[
            pl.BlockSpec((NC, blk, ROW), lambda i: (0, i, 0)),
            pl.BlockSpec((1, EMB), lambda i: (0, 0)),
        ],
        out_specs=pl.BlockSpec((blk, EMB), lambda i: (i, 0)),
        out_shape=jax.ShapeDtypeStruct((N, EMB), F32),
    )(uext, b_gat)


def _big_body(embi_ref, embj_ref, xa_ref, x_ref, s_ref):
    j = pl.program_id(1)
    s_ref[...] = jax.nn.sigmoid(
        lax.dot_general(embi_ref[...], embj_ref[...],
                        (((1,), (1,)), ((), ())), preferred_element_type=F32))

    @pl.when(j == 0)
    def _():
        x_ref[...] = lax.dot_general(embi_ref[...], xa_ref[...],
                                     (((1,), (1,)), ((), ())),
                                     preferred_element_type=F32)


def _tc_big(emb, xa):
    bi = 2000
    bj = 1024
    gi = N // bi
    gj = pl.cdiv(N, bj)
    return pl.pallas_call(
        _big_body,
        grid=(gi, gj),
        in_specs=[
            pl.BlockSpec((bi, EMB), lambda i, j: (i, 0)),
            pl.BlockSpec((bj, EMB), lambda i, j: (j, 0)),
            pl.BlockSpec((IN_DIM, HID), lambda i, j: (0, 0)),
        ],
        out_specs=[
            pl.BlockSpec((bi, IN_DIM), lambda i, j: (i, 0)),
            pl.BlockSpec((bi, bj), lambda i, j: (i, j)),
        ],
        out_shape=[
            jax.ShapeDtypeStruct((N, IN_DIM), F32),
            jax.ShapeDtypeStruct((N, N), F32),
        ],
    )(emb, emb, xa)


def kernel(x, edge_index, batch_size, W_stru, b_stru, W_gat, att_src,
           att_dst, b_gat, W_a1, b_a1, W_a2, b_a2):
    x = x.astype(F32)
    hp_ext, a_s2, a_d2, m_out, xa = _tc_encode(
        x, W_stru, b_stru.reshape(1, EMB), W_gat,
        att_src.reshape(HID, 1), att_dst.reshape(HID, 1),
        W_a1, b_a1.reshape(1, EMB), W_a2, b_a2.reshape(1, HID))

    loops = jnp.arange(N, dtype=jnp.int32)
    pad = jnp.zeros((E_PAD - E_TOT,), jnp.int32)
    src = jnp.concatenate(
        [edge_index[0].astype(jnp.int32), loops, pad]).reshape(NW, BPW, BLK)
    dst = jnp.concatenate(
        [edge_index[1].astype(jnp.int32), loops, pad]).reshape(NW, BPW, BLK)

    uext = _sc_edge(hp_ext, a_s2.reshape(N), a_d2.reshape(N), src, dst,
                    m_out[0, :16], jnp.zeros((ZR, ROW), F32))

    emb = _tc_mid(uext, b_gat.reshape(1, HID))
    x_, s_ = _tc_big(emb, xa)
    return (x_, s_)


# speedup vs baseline: 1.5520x; 1.0209x over previous
import functools

import jax
import jax.numpy as jnp
from jax import lax
from jax.experimental import pallas as pl
from jax.experimental.pallas import tpu as pltpu
from jax.experimental.pallas import tpu_sc as plsc

N = 10000
IN_DIM = 128
EMB = 64
HID = 64
E_IN = 160000
E_TOT = E_IN + N

NC = 2
NS = 16
NW = NC * NS
ROW = 80
BLK = 128
BPW = -(-E_TOT // (NW * BLK))
EPW = BPW * BLK
E_PAD = EPW * NW
ZR = 632
N_PAD = NS * ZR

F32 = jnp.float32


def _enc_body(x_ref, ws_ref, bs_ref, wg_ref, asrc_ref, adst_ref,
              wa1_ref, ba1_ref, wa2_ref, ba2_ref,
              hpext_ref, as_ref, ad_ref, m_ref, xa_ref, ms_ref, md_ref,
              acc_ref):
    i = pl.program_id(0)
    h = jnp.maximum(jnp.dot(x_ref[...], ws_ref[...],
                            preferred_element_type=F32) + bs_ref[...], 0.0)
    hp = jnp.dot(h, wg_ref[...], preferred_element_type=F32)
    a_s = jnp.dot(hp, asrc_ref[...], preferred_element_type=F32)
    a_d = jnp.dot(hp, adst_ref[...], preferred_element_type=F32)
    blk = hp.shape[0]
    ones = jnp.ones((blk, 1), F32)
    zeros = jnp.zeros((blk, ROW - EMB - 1), F32)
    hpext_ref[...] = jnp.concatenate([hp, ones, zeros], axis=1)
    as_ref[...] = a_s
    ad_ref[...] = a_d

    @pl.when(i == 0)
    def _():
        ms_ref[0] = -jnp.inf
        md_ref[0] = -jnp.inf

    ms_ref[0] = jnp.maximum(ms_ref[0], jnp.max(a_s))
    md_ref[0] = jnp.maximum(md_ref[0], jnp.max(a_d))
    m_ref[...] = jnp.full((1, 128), jnp.maximum(ms_ref[0] + md_ref[0], 0.0), F32)

    @pl.when(i == 0)
    def _():
        acc_ref[...] = jnp.zeros_like(acc_ref)

    acc_ref[...] += lax.dot_general(x_ref[...], wa1_ref[...],
                                    (((0,), (0,)), ((), ())),
                                    preferred_element_type=F32)

    @pl.when(i == pl.num_programs(0) - 1)
    def _():
        xa1 = jnp.maximum(acc_ref[...] + ba1_ref[...], 0.0)
        xa_ref[...] = jnp.dot(xa1, wa2_ref[...],
                              preferred_element_type=F32) + ba2_ref[...]


def _tc_encode(x, W_stru, b_stru, W_gat, att_src, att_dst,
               W_a1, b_a1, W_a2, b_a2):
    blk = 2000
    grid = N // blk
    return pl.pallas_call(
        _enc_body,
        grid=(grid,),
        in_specs=[
            pl.BlockSpec((blk, IN_DIM), lambda i: (i, 0)),
            pl.BlockSpec((IN_DIM, EMB), lambda i: (0, 0)),
            pl.BlockSpec((1, EMB), lambda i: (0, 0)),
            pl.BlockSpec((EMB, HID), lambda i: (0, 0)),
            pl.BlockSpec((HID, 1), lambda i: (0, 0)),
            pl.BlockSpec((HID, 1), lambda i: (0, 0)),
            pl.BlockSpec((blk, EMB), lambda i: (i, 0)),
            pl.BlockSpec((1, EMB), lambda i: (0, 0)),
            pl.BlockSpec((EMB, HID), lambda i: (0, 0)),
            pl.BlockSpec((1, HID), lambda i: (0, 0)),
        ],
        out_specs=[
            pl.BlockSpec((blk, ROW), lambda i: (i, 0)),
            pl.BlockSpec((blk, 1), lambda i: (i, 0)),
            pl.BlockSpec((blk, 1), lambda i: (i, 0)),
            pl.BlockSpec((1, 128), lambda i: (0, 0)),
            pl.BlockSpec((IN_DIM, HID), lambda i: (0, 0)),
        ],
        out_shape=[
            jax.ShapeDtypeStruct((N, ROW), F32),
            jax.ShapeDtypeStruct((N, 1), F32),
            jax.ShapeDtypeStruct((N, 1), F32),
            jax.ShapeDtypeStruct((1, 128), F32),
            jax.ShapeDtypeStruct((IN_DIM, HID), F32),
        ],
        scratch_shapes=[pltpu.SMEM((1,), F32), pltpu.SMEM((1,), F32),
                        pltpu.VMEM((IN_DIM, EMB), F32)],
    )(x, W_stru, b_stru, W_gat, att_src, att_dst, W_a1, b_a1, W_a2, b_a2)


NBUF = 3


def _sc_edge_body(hp_hbm, asrc_hbm, adst_hbm, src_hbm, dst_hbm, m_hbm, z_hbm,
                  out_hbm,
                  a_s_v, a_d_v, m_v, idx_s_all, idx_d_all,
                  r0, r1, r2, g0, g1, g2, s0, s1, s2, u_sh):
    cid = lax.axis_index("c")
    sid = lax.axis_index("s")
    wid = sid * NC + cid

    rows = (r0, r1, r2)
    gsem = (g0, g1, g2)
    ssem = (s0, s1, s2)

    pltpu.sync_copy(asrc_hbm, a_s_v)
    pltpu.sync_copy(adst_hbm, a_d_v)
    pltpu.sync_copy(m_hbm, m_v)
    pltpu.sync_copy(src_hbm.at[wid], idx_s_all)
    pltpu.sync_copy(dst_hbm.at[wid], idx_d_all)
    pltpu.sync_copy(z_hbm, u_sh.at[pl.ds(sid * ZR, ZR)])
    plsc.subcore_barrier()

    def start_gather(q, b):
        pltpu.async_copy(hp_hbm.at[idx_s_all.at[b]], rows[q], gsem[q])

    def drain_scatter(q):
        pltpu.make_async_copy(rows[q], u_sh.at[idx_d_all.at[0]], ssem[q]).wait()

    def process(q, b):
        pltpu.make_async_copy(hp_hbm.at[idx_s_all.at[b]], rows[q], gsem[q]).wait()
        base = wid * EPW + b * BLK
        m_vec = m_v[...]
        rq = rows[q]
        for g in range(BLK // 16):
            s_i = idx_s_all[b, pl.ds(g * 16, 16)]
            d_i = idx_d_all[b, pl.ds(g * 16, 16)]
            av = plsc.load_gather(a_s_v, [s_i])
            bv = plsc.load_gather(a_d_v, [d_i])
            e = av + bv
            e = jnp.where(e >= 0.0, e, e * jnp.float32(0.2))
            w = jnp.exp(e - m_vec)
            gids = base + g * 16 + lax.iota(jnp.int32, 16)
            w = jnp.where(gids < E_TOT, w, 0.0)
            for lane in range(16):
                i = g * 16 + lane
                ws = w[lane]
                for ch in range(ROW // 16):
                    rq[i, pl.ds(ch * 16, 16)] = rq[i, pl.ds(ch * 16, 16)] * ws
        pltpu.async_copy(rq, u_sh.at[idx_d_all.at[b]], ssem[q], add=True)

    start_gather(0, 0)
    start_gather(1, 1)

    def outer_body(i, carry):
        for q in range(NBUF):
            b = i * NBUF + q
            process(q, b)
            nq = (q + 2) % NBUF
            if q == 0:
                @pl.when(i > 0)
                def _():
                    drain_scatter(nq)

                start_gather(nq, b + 2)
            else:
                @pl.when(i < BPW // NBUF - 1)
                def _():
                    drain_scatter(nq)
                    start_gather(nq, b + 2)
        return carry

    lax.fori_loop(0, BPW // NBUF, outer_body, 0)
    for q in range(NBUF):
        drain_scatter(q)
    plsc.subcore_barrier()
    pltpu.sync_copy(u_sh.at[pl.ds(sid * ZR, ZR)],
                    out_hbm.at[cid, pl.ds(sid * ZR, ZR)])


def _sc_edge(hp_ext, a_src, a_dst, src, dst, m8, zeros_tile):
    mesh = plsc.VectorSubcoreMesh(core_axis_name="c", subcore_axis_name="s",
                                  num_cores=NC, num_subcores=NS)
    return pl.kernel(
        _sc_edge_body,
        out_type=jax.ShapeDtypeStruct((NC, N_PAD, ROW), F32),
        mesh=mesh,
        scratch_types=(
            [pltpu.VMEM((N,), F32),
             pltpu.VMEM((N,), F32),
             pltpu.VMEM((16,), F32)]
            + [pltpu.VMEM((BPW, BLK), jnp.int32)] * 2
            + [pltpu.VMEM((BLK, ROW), F32)] * NBUF
            + [pltpu.SemaphoreType.DMA] * (2 * NBUF)
            + [pltpu.VMEM_SHARED((N_PAD, ROW), F32)]
        ),
        compiler_params=pltpu.CompilerParams(needs_layout_passes=False, use_tc_tiling_on_sc=False),
    )(hp_ext, a_src, a_dst, src, dst, m8, zeros_tile)


def _mid_body(u_ref, bg_ref, emb_ref):
    u = u_ref[0] + u_ref[1]
    emb_ref[...] = u[:, :EMB] / (u[:, EMB:EMB + 1] + 1e-16) + bg_ref[...]


def _tc_mid(uext, b_gat):
    blk = 2000
    grid = N // blk
    return pl.pallas_call(
        _mid_body,
        grid=(grid,),
        in_specs=[
            pl.BlockSpec((NC, blk, ROW), lambda i: (0, i, 0)),
            pl.BlockSpec((1, EMB), lambda i: (0, 0)),
        ],
        out_specs=pl.BlockSpec((blk, EMB), lambda i: (i, 0)),
        out_shape=jax.ShapeDtypeStruct((N, EMB), F32),
    )(uext, b_gat)


def _big_body(embi_ref, embj_ref, xa_ref, x_ref, s_ref):
    j = pl.program_id(1)
    s_ref[...] = jax.nn.sigmoid(
        lax.dot_general(embi_ref[...], embj_ref[...],
                        (((1,), (1,)), ((), ())), preferred_element_type=F32))

    @pl.when(j == 0)
    def _():
        x_ref[...] = lax.dot_general(embi_ref[...], xa_ref[...],
                                     (((1,), (1,)), ((), ())),
                                     preferred_element_type=F32)


def _tc_big(emb, xa):
    bi = 2000
    bj = 2048
    gi = N // bi
    gj = pl.cdiv(N, bj)
    return pl.pallas_call(
        _big_body,
        grid=(gi, gj),
        in_specs=[
            pl.BlockSpec((bi, EMB), lambda i, j: (i, 0)),
            pl.BlockSpec((bj, EMB), lambda i, j: (j, 0)),
            pl.BlockSpec((IN_DIM, HID), lambda i, j: (0, 0)),
        ],
        out_specs=[
            pl.BlockSpec((bi, IN_DIM), lambda i, j: (i, 0)),
            pl.BlockSpec((bi, bj), lambda i, j: (i, j)),
        ],
        out_shape=[
            jax.ShapeDtypeStruct((N, IN_DIM), F32),
            jax.ShapeDtypeStruct((N, N), F32),
        ],
    )(emb, emb, xa)


def kernel(x, edge_index, batch_size, W_stru, b_stru, W_gat, att_src,
           att_dst, b_gat, W_a1, b_a1, W_a2, b_a2):
    x = x.astype(F32)
    hp_ext, a_s2, a_d2, m_out, xa = _tc_encode(
        x, W_stru, b_stru.reshape(1, EMB), W_gat,
        att_src.reshape(HID, 1), att_dst.reshape(HID, 1),
        W_a1, b_a1.reshape(1, EMB), W_a2, b_a2.reshape(1, HID))

    loops = jnp.arange(N, dtype=jnp.int32)
    pad = jnp.zeros((E_PAD - E_TOT,), jnp.int32)
    src = jnp.concatenate(
        [edge_index[0].astype(jnp.int32), loops, pad]).reshape(NW, BPW, BLK)
    dst = jnp.concatenate(
        [edge_index[1].astype(jnp.int32), loops, pad]).reshape(NW, BPW, BLK)

    uext = _sc_edge(hp_ext, a_s2.reshape(N), a_d2.reshape(N), src, dst,
                    m_out[0, :16], jnp.zeros((ZR, ROW), F32))

    emb = _tc_mid(uext, b_gat.reshape(1, HID))
    x_, s_ = _tc_big(emb, xa)
    return (x_, s_)
